# Initial kernel scaffold; baseline (speedup 1.0000x reference)
import jax
import jax.numpy as jnp
from jax.experimental import pallas as pl


def kernel(z, charge, batch, atom_emb, charge_emb):
    raise NotImplementedError("write your pallas kernel here")


# speedup vs baseline: 5.0457x; 5.0457x over previous
import functools

import jax
import jax.numpy as jnp
from jax import lax
from jax.experimental import pallas as pl
from jax.experimental.pallas import tpu as pltpu
from jax.experimental.pallas import tpu_sc as plsc

B = 1_000_000
H = 128
NE = 100
NC_CHG = 5
NMOL = 32768

NCORES = 2
NSUB = 16
NW = NCORES * NSUB

G = 80
K = 5
NB = G * K
NUM_BLOCKS = B // NB


def _fuse_body(a_ref, c_ref, o_ref):
    o_ref[...] = a_ref[...][:, None, :] + c_ref[...][None, :, :]


def _fused_table(atom_emb, charge_emb):
    out3 = pl.pallas_call(
        _fuse_body,
        out_shape=jax.ShapeDtypeStruct((NE, NC_CHG, H), jnp.float32),
    )(atom_emb, charge_emb)
    return out3.reshape(NE * NC_CHG, H)


def _sc_body(z_hbm, batch_hbm, charge_hbm, fused_hbm, out_hbm,
             chg_v, z_v, b_v, idx_v, rows_v, sem_g):
    wid = lax.axis_index("s") * NCORES + lax.axis_index("c")
    pltpu.sync_copy(charge_hbm, chg_v)

    n_blocks_w = (NUM_BLOCKS - wid + NW - 1) // NW

    def block_body(t, carry):
        bid = wid + t * NW
        base = bid * NB
        pltpu.sync_copy(z_hbm.at[pl.ds(base, NB)], z_v)
        pltpu.sync_copy(batch_hbm.at[pl.ds(base, NB)], b_v)
        for j in range(K):
            for m in range(G // 16):
                s = pl.ds(j * G + m * 16, 16)
                z16 = z_v[s]
                b16 = b_v[s]
                ch16 = plsc.load_gather(chg_v, [b16])
                idx_v[j, pl.ds(m * 16, 16)] = (z16 - 1) * NC_CHG + ch16
        copies = [
            pltpu.async_copy(fused_hbm.at[idx_v.at[j]], rows_v.at[j], sem_g)
            for j in range(K)
        ]
        for c in copies:
            c.wait()
        for j in range(K):
            pltpu.sync_copy(rows_v.at[j], out_hbm.at[pl.ds(base + j * G, G)])
        return carry

    lax.fori_loop(0, n_blocks_w, block_body, 0)


@jax.jit
def kernel(z, charge, batch, atom_emb, charge_emb):
    fused = _fused_table(atom_emb, charge_emb)
    mesh = plsc.VectorSubcoreMesh(core_axis_name="c", subcore_axis_name="s")
    sc = pl.kernel(
        _sc_body,
        out_type=jax.ShapeDtypeStruct((B, H), jnp.float32),
        mesh=mesh,
        compiler_params=pltpu.CompilerParams(needs_layout_passes=False),
        scratch_types=[
            pltpu.VMEM((NMOL,), jnp.int32),
            pltpu.VMEM((NB,), jnp.int32),
            pltpu.VMEM((NB,), jnp.int32),
            pltpu.VMEM((K, G), jnp.int32),
            pltpu.VMEM((K, G, H), jnp.float32),
            pltpu.SemaphoreType.DMA,
        ],
    )
    return sc(z.astype(jnp.int32), batch.astype(jnp.int32),
              charge.astype(jnp.int32), fused)


# speedup vs baseline: 18.8960x; 3.7450x over previous
import functools

import jax
import jax.numpy as jnp
from jax import lax
from jax.experimental import pallas as pl
from jax.experimental.pallas import tpu as pltpu
from jax.experimental.pallas import tpu_sc as plsc

B = 1_000_000
H = 128
NE = 100
NC_CHG = 5
NMOL = 32768

NCORES = 2
NSUB = 16
NW = NCORES * NSUB

G = 80
K = 4
NB = G * K
NUM_BLOCKS = B // NB
SLOTS = (NUM_BLOCKS + NW - 1) // NW


def _fuse_body(a_ref, c_ref, o_ref):
    o_ref[...] = a_ref[...][:, None, :] + c_ref[...][None, :, :]


def _fused_table(atom_emb, charge_emb):
    out3 = pl.pallas_call(
        _fuse_body,
        out_shape=jax.ShapeDtypeStruct((NE, NC_CHG, H), jnp.float32),
    )(atom_emb, charge_emb)
    return out3.reshape(NE * NC_CHG, H)


def _sc_body(z_hbm, batch_hbm, charge_hbm, fused_hbm, out_hbm,
             chg_v, z_v0, z_v1, b_v0, b_v1, idx_v0, idx_v1,
             rows_v0, rows_v1, fused_sh,
             sem_g0, sem_g1, sem_o0, sem_o1, sem_i0, sem_i1):
    wid = lax.axis_index("s") * NCORES + lax.axis_index("c")
    z_v = (z_v0, z_v1)
    b_v = (b_v0, b_v1)
    idx_v = (idx_v0, idx_v1)
    rows_v = (rows_v0, rows_v1)
    sem_g = (sem_g0, sem_g1)
    sem_o = (sem_o0, sem_o1)
    sem_i = (sem_i0, sem_i1)

    @pl.when(lax.axis_index("s") == 0)
    def _():
        pltpu.sync_copy(fused_hbm, fused_sh)

    pltpu.sync_copy(charge_hbm, chg_v)
    plsc.subcore_barrier()

    def bid_of(t):
        return jnp.minimum(wid + t * NW, NUM_BLOCKS - 1)

    def fire_i(t, par):
        base = bid_of(t) * NB
        pltpu.async_copy(z_hbm.at[pl.ds(base, NB)], z_v[par], sem_i[par])
        pltpu.async_copy(batch_hbm.at[pl.ds(base, NB)], b_v[par],
                         sem_i[par])

    def compute(t, par):
        base = bid_of(t) * NB
        pltpu.make_async_copy(z_hbm.at[pl.ds(base, NB)], z_v[par],
                              sem_i[par]).wait()
        pltpu.make_async_copy(batch_hbm.at[pl.ds(base, NB)], b_v[par],
                              sem_i[par]).wait()
        for j in range(K):
            for m in range(G // 16):
                s = pl.ds(j * G + m * 16, 16)
                z16 = z_v[par][s]
                b16 = b_v[par][s]
                ch16 = plsc.load_gather(chg_v, [b16])
                idx_v[par][j, pl.ds(m * 16, 16)] = (z16 - 1) * NC_CHG + ch16

    def fire_g(par):
        for j in range(K):
            pltpu.async_copy(fused_sh.at[idx_v[par].at[j]],
                             rows_v[par].at[j], sem_g[par])

    def wait_g(par):
        for j in range(K):
            pltpu.make_async_copy(fused_sh.at[idx_v[par].at[j]],
                                  rows_v[par].at[j], sem_g[par]).wait()

    def fire_w(t, par):
        base = bid_of(t) * NB
        for j in range(K):
            pltpu.async_copy(rows_v[par].at[j],
                             out_hbm.at[pl.ds(base + j * G, G)], sem_o[par])

    def wait_w(t, par):
        base = bid_of(t) * NB
        for j in range(K):
            pltpu.make_async_copy(rows_v[par].at[j],
                                  out_hbm.at[pl.ds(base + j * G, G)],
                                  sem_o[par]).wait()

    fire_i(0, 0)
    fire_i(1, 1)
    compute(0, 0)
    fire_g(0)
    fire_i(2, 0)
    compute(1, 1)
    fire_g(1)
    fire_i(3, 1)
    wait_g(0)
    fire_w(0, 0)
    compute(2, 0)

    def loop_body(q, carry):
        t0 = 2 * q
        t1 = t0 + 1
        wait_w(t0 - 2, 0)
        fire_g(0)
        fire_i(t0 + 2, 0)
        wait_g(1)
        fire_w(t0 - 1, 1)
        compute(t0 + 1, 1)
        wait_w(t1 - 2, 1)
        fire_g(1)
        fire_i(t1 + 2, 1)
        wait_g(0)
        fire_w(t1 - 1, 0)
        compute(t1 + 1, 0)
        return carry

    lax.fori_loop(1, SLOTS // 2, loop_body, 0)

    wait_g(1)
    fire_w(SLOTS - 1, 1)
    wait_w(SLOTS - 2, 0)
    wait_w(SLOTS - 1, 1)


@jax.jit
def kernel(z, charge, batch, atom_emb, charge_emb):
    fused = _fused_table(atom_emb, charge_emb)
    mesh = plsc.VectorSubcoreMesh(core_axis_name="c", subcore_axis_name="s")
    sc = pl.kernel(
        _sc_body,
        out_type=jax.ShapeDtypeStruct((B, H), jnp.float32),
        mesh=mesh,
        compiler_params=pltpu.CompilerParams(needs_layout_passes=False),
        scratch_types=[
            pltpu.VMEM((NMOL,), jnp.int32),
            pltpu.VMEM((NB,), jnp.int32),
            pltpu.VMEM((NB,), jnp.int32),
            pltpu.VMEM((NB,), jnp.int32),
            pltpu.VMEM((NB,), jnp.int32),
            pltpu.VMEM((K, G), jnp.int32),
            pltpu.VMEM((K, G), jnp.int32),
            pltpu.VMEM((K, G, H), jnp.float32),
            pltpu.VMEM((K, G, H), jnp.float32),
            pltpu.VMEM_SHARED((NE * NC_CHG, H), jnp.float32),
            pltpu.SemaphoreType.DMA,
            pltpu.SemaphoreType.DMA,
            pltpu.SemaphoreType.DMA,
            pltpu.SemaphoreType.DMA,
            pltpu.SemaphoreType.DMA,
            pltpu.SemaphoreType.DMA,
        ],
    )
    return sc(z.astype(jnp.int32), batch.astype(jnp.int32),
              charge.astype(jnp.int32), fused)


# speedup vs baseline: 18.9238x; 1.0015x over previous
import functools

import jax
import jax.numpy as jnp
from jax import lax
from jax.experimental import pallas as pl
from jax.experimental.pallas import tpu as pltpu
from jax.experimental.pallas import tpu_sc as plsc

B = 1_000_000
H = 128
NE = 100
NC_CHG = 5
NMOL = 32768

NCORES = 2
NSUB = 16
NW = NCORES * NSUB

G = 80
K = 4
NB = G * K
NUM_BLOCKS = B // NB
SLOTS = (NUM_BLOCKS + NW - 1) // NW


def _fuse_body(a_ref, c_ref, o_ref):
    o_ref[...] = a_ref[...][:, None, :] + c_ref[...][None, :, :]


def _fused_table(atom_emb, charge_emb):
    out3 = pl.pallas_call(
        _fuse_body,
        out_shape=jax.ShapeDtypeStruct((NE, NC_CHG, H), jnp.float32),
    )(atom_emb, charge_emb)
    return out3.reshape(NE * NC_CHG, H)


def _sc_body(z_hbm, batch_hbm, charge_hbm, fused_hbm, out_hbm,
             chg_v, z_v0, z_v1, b_v0, b_v1, idx_v0, idx_v1,
             rows_v0, rows_v1, fused_sh,
             sem_g0, sem_g1, sem_o0, sem_o1, sem_i0, sem_i1):
    wid = lax.axis_index("s") * NCORES + lax.axis_index("c")
    z_v = (z_v0, z_v1)
    b_v = (b_v0, b_v1)
    idx_v = (idx_v0, idx_v1)
    rows_v = (rows_v0, rows_v1)
    sem_g = (sem_g0, sem_g1)
    sem_o = (sem_o0, sem_o1)
    sem_i = (sem_i0, sem_i1)

    @pl.when(lax.axis_index("s") == 0)
    def _():
        pltpu.sync_copy(fused_hbm, fused_sh)

    pltpu.sync_copy(charge_hbm, chg_v)
    plsc.subcore_barrier()

    def bid_of(t):
        return jnp.minimum(wid + t * NW, NUM_BLOCKS - 1)

    def fire_i(t, par):
        base = bid_of(t) * NB
        pltpu.async_copy(z_hbm.at[pl.ds(base, NB)], z_v[par], sem_i[par])
        pltpu.async_copy(batch_hbm.at[pl.ds(base, NB)], b_v[par],
                         sem_i[par])

    def compute(t, par):
        base = bid_of(t) * NB
        pltpu.make_async_copy(z_hbm.at[pl.ds(base, NB)], z_v[par],
                              sem_i[par]).wait()
        pltpu.make_async_copy(batch_hbm.at[pl.ds(base, NB)], b_v[par],
                              sem_i[par]).wait()
        for j in range(K):
            for m in range(G // 16):
                s = pl.ds(j * G + m * 16, 16)
                z16 = z_v[par][s]
                b16 = b_v[par][s]
                ch16 = plsc.load_gather(chg_v, [b16])
                idx_v[par][j, pl.ds(m * 16, 16)] = (z16 - 1) * NC_CHG + ch16

    def fire_g(par):
        for j in range(K):
            pltpu.async_copy(fused_sh.at[idx_v[par].at[j]],
                             rows_v[par].at[pl.ds(j * G, G)], sem_g[par])

    def wait_g(par):
        for j in range(K):
            pltpu.make_async_copy(fused_sh.at[idx_v[par].at[j]],
                                  rows_v[par].at[pl.ds(j * G, G)],
                                  sem_g[par]).wait()

    def fire_w(t, par):
        base = bid_of(t) * NB
        pltpu.async_copy(rows_v[par], out_hbm.at[pl.ds(base, NB)],
                         sem_o[par])

    def wait_w(t, par):
        base = bid_of(t) * NB
        pltpu.make_async_copy(rows_v[par], out_hbm.at[pl.ds(base, NB)],
                              sem_o[par]).wait()

    fire_i(0, 0)
    fire_i(1, 1)
    compute(0, 0)
    fire_g(0)
    fire_i(2, 0)
    compute(1, 1)
    fire_g(1)
    fire_i(3, 1)
    wait_g(0)
    fire_w(0, 0)
    compute(2, 0)

    def loop_body(q, carry):
        t0 = 2 * q
        t1 = t0 + 1
        wait_w(t0 - 2, 0)
        fire_g(0)
        fire_i(t0 + 2, 0)
        wait_g(1)
        fire_w(t0 - 1, 1)
        compute(t0 + 1, 1)
        wait_w(t1 - 2, 1)
        fire_g(1)
        fire_i(t1 + 2, 1)
        wait_g(0)
        fire_w(t1 - 1, 0)
        compute(t1 + 1, 0)
        return carry

    lax.fori_loop(1, SLOTS // 2, loop_body, 0)

    wait_g(1)
    fire_w(SLOTS - 1, 1)
    wait_w(SLOTS - 2, 0)
    wait_w(SLOTS - 1, 1)


@jax.jit
def kernel(z, charge, batch, atom_emb, charge_emb):
    fused = _fused_table(atom_emb, charge_emb)
    mesh = plsc.VectorSubcoreMesh(core_axis_name="c", subcore_axis_name="s")
    sc = pl.kernel(
        _sc_body,
        out_type=jax.ShapeDtypeStruct((B, H), jnp.float32),
        mesh=mesh,
        compiler_params=pltpu.CompilerParams(needs_layout_passes=False),
        scratch_types=[
            pltpu.VMEM((NMOL,), jnp.int32),
            pltpu.VMEM((NB,), jnp.int32),
            pltpu.VMEM((NB,), jnp.int32),
            pltpu.VMEM((NB,), jnp.int32),
            pltpu.VMEM((NB,), jnp.int32),
            pltpu.VMEM((K, G), jnp.int32),
            pltpu.VMEM((K, G), jnp.int32),
            pltpu.VMEM((NB, H), jnp.float32),
            pltpu.VMEM((NB, H), jnp.float32),
            pltpu.VMEM_SHARED((NE * NC_CHG, H), jnp.float32),
            pltpu.SemaphoreType.DMA,
            pltpu.SemaphoreType.DMA,
            pltpu.SemaphoreType.DMA,
            pltpu.SemaphoreType.DMA,
            pltpu.SemaphoreType.DMA,
            pltpu.SemaphoreType.DMA,
        ],
    )
    return sc(z.astype(jnp.int32), batch.astype(jnp.int32),
              charge.astype(jnp.int32), fused)


# speedup vs baseline: 19.3962x; 1.0250x over previous
import jax
import jax.numpy as jnp
from jax import lax
from jax.experimental import pallas as pl
from jax.experimental.pallas import tpu as pltpu
from jax.experimental.pallas import tpu_sc as plsc

B = 1_000_000
H = 128
NE = 100
NC_CHG = 5
NMOL = 32768

NCORES = 2
NSUB = 16
NW = NCORES * NSUB

NB = 80
D = 4
NUM_BLOCKS = B // NB
SLOTS = 392


def _fuse_body(a_ref, c_ref, o_ref):
    o_ref[...] = a_ref[...][:, None, :] + c_ref[...][None, :, :]


def _fused_table(atom_emb, charge_emb):
    out3 = pl.pallas_call(
        _fuse_body,
        out_shape=jax.ShapeDtypeStruct((NE, NC_CHG, H), jnp.float32),
    )(atom_emb, charge_emb)
    return out3.reshape(NE * NC_CHG, H)


def _sc_body(z_hbm, batch_hbm, charge_hbm, fused_hbm, out_hbm,
             chg_v, z_v0, z_v1, z_v2, z_v3, b_v0, b_v1, b_v2, b_v3,
             idx_v0, idx_v1, idx_v2, idx_v3,
             rows_v0, rows_v1, rows_v2, rows_v3, fused_sh,
             sem_g0, sem_g1, sem_g2, sem_g3,
             sem_o0, sem_o1, sem_o2, sem_o3,
             sem_i0, sem_i1, sem_i2, sem_i3):
    wid = lax.axis_index("s") * NCORES + lax.axis_index("c")
    z_v = (z_v0, z_v1, z_v2, z_v3)
    b_v = (b_v0, b_v1, b_v2, b_v3)
    idx_v = (idx_v0, idx_v1, idx_v2, idx_v3)
    rows_v = (rows_v0, rows_v1, rows_v2, rows_v3)
    sem_g = (sem_g0, sem_g1, sem_g2, sem_g3)
    sem_o = (sem_o0, sem_o1, sem_o2, sem_o3)
    sem_i = (sem_i0, sem_i1, sem_i2, sem_i3)

    @pl.when(lax.axis_index("s") == 0)
    def _():
        pltpu.sync_copy(fused_hbm, fused_sh)

    pltpu.sync_copy(charge_hbm, chg_v)
    plsc.subcore_barrier()

    def bid_of(t):
        return jnp.minimum(wid + t * NW, NUM_BLOCKS - 1)

    def fire_i(t, par):
        base = bid_of(t) * NB
        pltpu.async_copy(z_hbm.at[pl.ds(base, NB)], z_v[par], sem_i[par])
        pltpu.async_copy(batch_hbm.at[pl.ds(base, NB)], b_v[par],
                         sem_i[par])

    def compute(t, par):
        base = bid_of(t) * NB
        pltpu.make_async_copy(z_hbm.at[pl.ds(base, NB)], z_v[par],
                              sem_i[par]).wait()
        pltpu.make_async_copy(batch_hbm.at[pl.ds(base, NB)], b_v[par],
                              sem_i[par]).wait()
        for m in range(NB // 16):
            s = pl.ds(m * 16, 16)
            z16 = z_v[par][s]
            b16 = b_v[par][s]
            ch16 = plsc.load_gather(chg_v, [b16])
            idx_v[par][s] = (z16 - 1) * NC_CHG + ch16

    def fire_g(par):
        pltpu.async_copy(fused_sh.at[idx_v[par]], rows_v[par], sem_g[par])

    def wait_g(par):
        pltpu.make_async_copy(fused_sh.at[idx_v[par]], rows_v[par],
                              sem_g[par]).wait()

    def fire_w(t, par):
        base = bid_of(t) * NB
        pltpu.async_copy(rows_v[par], out_hbm.at[pl.ds(base, NB)],
                         sem_o[par])

    def wait_w(t, par):
        base = bid_of(t) * NB
        pltpu.make_async_copy(rows_v[par], out_hbm.at[pl.ds(base, NB)],
                              sem_o[par]).wait()

    fire_i(0, 0)
    fire_i(1, 1)
    compute(0, 0)
    fire_g(0)
    fire_i(2, 2)
    compute(1, 1)
    fire_g(1)
    fire_i(3, 3)
    wait_g(0)
    fire_w(0, 0)
    compute(2, 2)
    fire_g(2)
    fire_i(4, 0)
    wait_g(1)
    fire_w(1, 1)
    compute(3, 3)
    fire_g(3)
    fire_i(5, 1)
    wait_g(2)
    fire_w(2, 2)
    compute(4, 0)

    def loop_body(q, carry):
        for r in range(D):
            t = 4 * q + r
            par = r
            wait_w(t - 4, par)
            fire_g(par)
            fire_i(t + 2, (r + 2) % D)
            wait_g((r - 1) % D)
            fire_w(t - 1, (r - 1) % D)
            compute(t + 1, (r + 1) % D)
        return carry

    lax.fori_loop(1, SLOTS // D, loop_body, 0)

    wait_g(3)
    fire_w(SLOTS - 1, 3)
    wait_w(SLOTS - 4, 0)
    wait_w(SLOTS - 3, 1)
    wait_w(SLOTS - 2, 2)
    wait_w(SLOTS - 1, 3)


@jax.jit
def kernel(z, charge, batch, atom_emb, charge_emb):
    fused = _fused_table(atom_emb, charge_emb)
    mesh = plsc.VectorSubcoreMesh(core_axis_name="c", subcore_axis_name="s")
    sc = pl.kernel(
        _sc_body,
        out_type=jax.ShapeDtypeStruct((B, H), jnp.float32),
        mesh=mesh,
        compiler_params=pltpu.CompilerParams(needs_layout_passes=False),
        scratch_types=(
            [pltpu.VMEM((NMOL,), jnp.int32)]
            + [pltpu.VMEM((NB,), jnp.int32) for _ in range(8)]
            + [pltpu.VMEM((NB,), jnp.int32) for _ in range(4)]
            + [pltpu.VMEM((NB, H), jnp.float32) for _ in range(4)]
            + [pltpu.VMEM_SHARED((NE * NC_CHG, H), jnp.float32)]
            + [pltpu.SemaphoreType.DMA for _ in range(12)]
        ),
    )
    return sc(z.astype(jnp.int32), batch.astype(jnp.int32),
              charge.astype(jnp.int32), fused)


# speedup vs baseline: 19.5311x; 1.0070x over previous
import jax
import jax.numpy as jnp
from jax import lax
from jax.experimental import pallas as pl
from jax.experimental.pallas import tpu as pltpu
from jax.experimental.pallas import tpu_sc as plsc

B = 1_000_000
H = 128
NE = 100
NC_CHG = 5
NMOL = 32768

NCORES = 2
NSUB = 16
NW = NCORES * NSUB

NB = 80
D = 4
NUM_BLOCKS = B // NB
SLOTS = 392


def _fuse_body(a_ref, c_ref, o_ref):
    o_ref[...] = a_ref[...][:, None, :] + c_ref[...][None, :, :]


def _fused_table(atom_emb, charge_emb):
    out3 = pl.pallas_call(
        _fuse_body,
        out_shape=jax.ShapeDtypeStruct((NE, NC_CHG, H), jnp.float32),
    )(atom_emb, charge_emb)
    return out3.reshape(NE * NC_CHG, H)


def _sc_body(z_hbm, batch_hbm, charge_hbm, fused_hbm, out_hbm,
             chg_v, z_v0, z_v1, z_v2, z_v3, b_v0, b_v1, b_v2, b_v3,
             idx_v0, idx_v1, idx_v2, idx_v3,
             rows_v0, rows_v1, rows_v2, rows_v3, fused_sh,
             sem_g0, sem_g1, sem_g2, sem_g3,
             sem_o0, sem_o1, sem_o2, sem_o3,
             sem_i0, sem_i1, sem_i2, sem_i3):
    wid = lax.axis_index("s") * NCORES + lax.axis_index("c")
    z_v = (z_v0, z_v1, z_v2, z_v3)
    b_v = (b_v0, b_v1, b_v2, b_v3)
    idx_v = (idx_v0, idx_v1, idx_v2, idx_v3)
    rows_v = (rows_v0, rows_v1, rows_v2, rows_v3)
    sem_g = (sem_g0, sem_g1, sem_g2, sem_g3)
    sem_o = (sem_o0, sem_o1, sem_o2, sem_o3)
    sem_i = (sem_i0, sem_i1, sem_i2, sem_i3)

    @pl.when(lax.axis_index("s") == 0)
    def _():
        pltpu.sync_copy(fused_hbm, fused_sh)

    pltpu.sync_copy(charge_hbm, chg_v)
    plsc.subcore_barrier()

    def bid_of(t):
        return jnp.minimum(wid + t * NW, NUM_BLOCKS - 1)

    def fire_i(t, par):
        base = bid_of(t) * NB
        pltpu.async_copy(z_hbm.at[pl.ds(base, NB)], z_v[par], sem_i[par])
        pltpu.async_copy(batch_hbm.at[pl.ds(base, NB)], b_v[par],
                         sem_i[par])

    def compute(t, par):
        base = bid_of(t) * NB
        pltpu.make_async_copy(z_hbm.at[pl.ds(base, NB)], z_v[par],
                              sem_i[par]).wait()
        pltpu.make_async_copy(batch_hbm.at[pl.ds(base, NB)], b_v[par],
                              sem_i[par]).wait()
        for m in range(NB // 16):
            s = pl.ds(m * 16, 16)
            z16 = z_v[par][s]
            b16 = b_v[par][s]
            ch16 = plsc.load_gather(chg_v, [b16])
            idx_v[par][s] = (z16 - 1) * NC_CHG + ch16

    def fire_g(par):
        pltpu.async_copy(fused_sh.at[idx_v[par]], rows_v[par], sem_g[par])

    def wait_g(par):
        pltpu.make_async_copy(fused_sh.at[idx_v[par]], rows_v[par],
                              sem_g[par]).wait()

    def fire_w(t, par):
        base = bid_of(t) * NB
        pltpu.async_copy(rows_v[par], out_hbm.at[pl.ds(base, NB)],
                         sem_o[par])

    def wait_w(t, par):
        base = bid_of(t) * NB
        pltpu.make_async_copy(rows_v[par], out_hbm.at[pl.ds(base, NB)],
                              sem_o[par]).wait()

    fire_i(0, 0)
    fire_i(1, 1)
    compute(0, 0)
    fire_g(0)
    fire_i(2, 2)
    compute(1, 1)
    fire_g(1)
    fire_i(3, 3)
    compute(2, 2)
    fire_g(2)
    fire_i(4, 0)
    wait_g(0)
    fire_w(0, 0)
    compute(3, 3)
    fire_g(3)
    fire_i(5, 1)
    wait_g(1)
    fire_w(1, 1)
    compute(4, 0)

    def loop_body(q, carry):
        for r in range(D):
            t = 4 * q + r
            par = r
            wait_w(t - 4, par)
            fire_g(par)
            fire_i(t + 2, (r + 2) % D)
            wait_g((r - 2) % D)
            fire_w(t - 2, (r - 2) % D)
            compute(t + 1, (r + 1) % D)
        return carry

    lax.fori_loop(1, SLOTS // D, loop_body, 0)

    wait_g((SLOTS - 2) % D)
    fire_w(SLOTS - 2, (SLOTS - 2) % D)
    wait_g((SLOTS - 1) % D)
    fire_w(SLOTS - 1, (SLOTS - 1) % D)
    wait_w(SLOTS - 4, 0)
    wait_w(SLOTS - 3, 1)
    wait_w(SLOTS - 2, 2)
    wait_w(SLOTS - 1, 3)


@jax.jit
def kernel(z, charge, batch, atom_emb, charge_emb):
    fused = _fused_table(atom_emb, charge_emb)
    mesh = plsc.VectorSubcoreMesh(core_axis_name="c", subcore_axis_name="s")
    sc = pl.kernel(
        _sc_body,
        out_type=jax.ShapeDtypeStruct((B, H), jnp.float32),
        mesh=mesh,
        compiler_params=pltpu.CompilerParams(needs_layout_passes=False),
        scratch_types=(
            [pltpu.VMEM((NMOL,), jnp.int32)]
            + [pltpu.VMEM((NB,), jnp.int32) for _ in range(8)]
            + [pltpu.VMEM((NB,), jnp.int32) for _ in range(4)]
            + [pltpu.VMEM((NB, H), jnp.float32) for _ in range(4)]
            + [pltpu.VMEM_SHARED((NE * NC_CHG, H), jnp.float32)]
            + [pltpu.SemaphoreType.DMA for _ in range(12)]
        ),
    )
    return sc(z.astype(jnp.int32), batch.astype(jnp.int32),
              charge.astype(jnp.int32), fused)
